# baseline (speedup 1.0000x reference)
import functools

import jax
import jax.numpy as jnp
from jax.experimental import pallas as pl
from jax.experimental.pallas import tpu as pltpu

_B, _N, _C = 2, 200000, 16
_D, _H, _W = 256, 256, 32
_BOUNDS = ((-48.0, 48.0), (-48.0, 48.0), (-4.0, 1.5))
_V = _D * _H * _W
_CH = 128
_VB = 2048
_NB = _V // _VB
_ROWS = 24
_ONES_ROW = 16
_KEY_ROW = 17
_NPAD = ((_N + _CH - 1) // _CH) * _CH


def _keys_kernel(pts_ref, keys_ref):
    (x0, x1), (y0, y1), (z0, z1) = _BOUNDS
    x = pts_ref[0, 0:1, :]
    y = pts_ref[0, 1:2, :]
    z = pts_ref[0, 2:3, :]
    vx = jnp.clip(((x - x0) / (x1 - x0) * _D).astype(jnp.int32), 0, _D - 1)
    vy = jnp.clip(((y - y0) / (y1 - y0) * _H).astype(jnp.int32), 0, _H - 1)
    vz = jnp.clip(((z - z0) / (z1 - z0) * _W).astype(jnp.int32), 0, _W - 1)
    keys_ref[0] = (vx * _H + vy) * _W + vz


def _main_kernel(starts_ref, fts_ref, out_ref, acc_ref):
    b = pl.program_id(0)
    j = pl.program_id(1)
    start = starts_ref[b, j]
    end = starts_ref[b, j + 1]
    c0 = start // _CH
    nchunks = (end + _CH - 1) // _CH - c0
    base = j * _VB

    acc_ref[...] = jnp.zeros((_ROWS, _VB), jnp.float32)

    def body(i, carry):
        off = pl.multiple_of((c0 + i) * _CH, _CH)
        chunk = fts_ref[0, :, pl.ds(off, _CH)]
        rel = chunk[_KEY_ROW:_KEY_ROW + 1, :].astype(jnp.int32) - base
        kcol = jnp.transpose(rel, (1, 0))
        oh = (kcol == jax.lax.broadcasted_iota(jnp.int32, (_CH, _VB), 1)
              ).astype(jnp.float32)
        acc_ref[...] += jnp.dot(chunk, oh,
                                preferred_element_type=jnp.float32)
        return carry

    jax.lax.fori_loop(0, nchunks, body, 0)

    acc = acc_ref[...]
    sums = acc[0:_C, :]
    cnt = acc[_ONES_ROW:_ONES_ROW + 1, :]
    out_ref[0] = jnp.where(cnt > 0.0, sums / jnp.maximum(cnt, 1.0), 0.0)


@jax.jit
def kernel(points, features):
    pts_t = jnp.swapaxes(points, 1, 2)
    keys = pl.pallas_call(
        _keys_kernel,
        out_shape=jax.ShapeDtypeStruct((_B, 1, _N), jnp.int32),
        grid=(_B,),
        in_specs=[pl.BlockSpec((1, 3, _N), lambda b: (b, 0, 0))],
        out_specs=pl.BlockSpec((1, 1, _N), lambda b: (b, 0, 0)),
        compiler_params=pltpu.CompilerParams(
            dimension_semantics=("parallel",)),
    )(pts_t)[:, 0, :]

    perm0 = jax.lax.broadcasted_iota(jnp.int32, (_B, _N), 1)
    skeys, perm = jax.lax.sort_key_val(keys, perm0)
    sfts = jnp.take_along_axis(features, perm[:, :, None], axis=1)

    packed = jnp.concatenate(
        [jnp.swapaxes(sfts, 1, 2),
         jnp.ones((_B, 1, _N), jnp.float32),
         skeys[:, None, :].astype(jnp.float32),
         jnp.zeros((_B, _ROWS - _KEY_ROW - 1, _N), jnp.float32)],
        axis=1)
    packed = jnp.pad(packed, ((0, 0), (0, 0), (0, _NPAD - _N)))

    bounds = (jnp.arange(_NB + 1, dtype=jnp.int32) * _VB)
    starts = jax.vmap(
        lambda a: jnp.searchsorted(a, bounds, side="left"))(skeys)
    starts = starts.astype(jnp.int32)

    out = pl.pallas_call(
        _main_kernel,
        out_shape=jax.ShapeDtypeStruct((_B, _C, _V), jnp.float32),
        grid_spec=pltpu.PrefetchScalarGridSpec(
            num_scalar_prefetch=1,
            grid=(_B, _NB),
            in_specs=[pl.BlockSpec((1, _ROWS, _NPAD),
                                   lambda b, j, s: (b, 0, 0))],
            out_specs=pl.BlockSpec((1, _C, _VB), lambda b, j, s: (b, 0, j)),
            scratch_shapes=[pltpu.VMEM((_ROWS, _VB), jnp.float32)],
        ),
        compiler_params=pltpu.CompilerParams(
            dimension_semantics=("parallel", "arbitrary"),
            vmem_limit_bytes=56 * 1024 * 1024),
    )(starts, packed)

    return out.reshape(_B, _C, _D, _H, _W)


# speedup vs baseline: 1.0518x; 1.0518x over previous
import functools

import jax
import jax.numpy as jnp
from jax.experimental import pallas as pl
from jax.experimental.pallas import tpu as pltpu

_B, _N, _C = 2, 200000, 16
_D, _H, _W = 256, 256, 32
_BOUNDS = ((-48.0, 48.0), (-48.0, 48.0), (-4.0, 1.5))
_V = _D * _H * _W
_CH = 128
_VB = 2048
_NB = _V // _VB
_ROWS = 24
_ONES_ROW = 16
_KEY_ROW = 17
_NPAD = ((_N + _CH - 1) // _CH) * _CH


def _keys_kernel(pts_ref, keys_ref):
    (x0, x1), (y0, y1), (z0, z1) = _BOUNDS
    x = pts_ref[0, 0:1, :]
    y = pts_ref[0, 1:2, :]
    z = pts_ref[0, 2:3, :]
    vx = jnp.clip(((x - x0) / (x1 - x0) * _D).astype(jnp.int32), 0, _D - 1)
    vy = jnp.clip(((y - y0) / (y1 - y0) * _H).astype(jnp.int32), 0, _H - 1)
    vz = jnp.clip(((z - z0) / (z1 - z0) * _W).astype(jnp.int32), 0, _W - 1)
    keys_ref[0] = (vx * _H + vy) * _W + vz


def _main_kernel(starts_ref, fts_ref, out_ref, acc_ref):
    b = pl.program_id(0)
    j = pl.program_id(1)
    start = starts_ref[b, j]
    end = starts_ref[b, j + 1]
    c0 = start // _CH
    nchunks = (end + _CH - 1) // _CH - c0
    base = j * _VB

    acc_ref[...] = jnp.zeros((_ROWS, _VB), jnp.float32)

    def body(i, carry):
        off = pl.multiple_of((c0 + i) * _CH, _CH)
        chunk = fts_ref[0, :, pl.ds(off, _CH)]
        rel = chunk[_KEY_ROW:_KEY_ROW + 1, :].astype(jnp.int32) - base
        oh = (jax.lax.broadcasted_iota(jnp.int32, (_VB, _CH), 0) == rel
              ).astype(jnp.float32)
        acc_ref[...] += jax.lax.dot_general(
            chunk, oh, (((1,), (1,)), ((), ())),
            preferred_element_type=jnp.float32)
        return carry

    jax.lax.fori_loop(0, nchunks, body, 0)

    acc = acc_ref[...]
    sums = acc[0:_C, :]
    cnt = acc[_ONES_ROW:_ONES_ROW + 1, :]
    out_ref[0] = jnp.where(cnt > 0.0, sums / jnp.maximum(cnt, 1.0), 0.0)


@jax.jit
def kernel(points, features):
    pts_t = jnp.swapaxes(points, 1, 2)
    keys = pl.pallas_call(
        _keys_kernel,
        out_shape=jax.ShapeDtypeStruct((_B, 1, _N), jnp.int32),
        grid=(_B,),
        in_specs=[pl.BlockSpec((1, 3, _N), lambda b: (b, 0, 0))],
        out_specs=pl.BlockSpec((1, 1, _N), lambda b: (b, 0, 0)),
        compiler_params=pltpu.CompilerParams(
            dimension_semantics=("parallel",)),
    )(pts_t)[:, 0, :]

    perm0 = jax.lax.broadcasted_iota(jnp.int32, (_B, _N), 1)
    skeys, perm = jax.lax.sort_key_val(keys, perm0)
    sfts = jnp.take_along_axis(features, perm[:, :, None], axis=1)

    packed = jnp.concatenate(
        [jnp.swapaxes(sfts, 1, 2),
         jnp.ones((_B, 1, _N), jnp.float32),
         skeys[:, None, :].astype(jnp.float32),
         jnp.zeros((_B, _ROWS - _KEY_ROW - 1, _N), jnp.float32)],
        axis=1)
    packed = jnp.pad(packed, ((0, 0), (0, 0), (0, _NPAD - _N)))

    bounds = (jnp.arange(_NB + 1, dtype=jnp.int32) * _VB)
    starts = jax.vmap(
        lambda a: jnp.searchsorted(a, bounds, side="left"))(skeys)
    starts = starts.astype(jnp.int32)

    out = pl.pallas_call(
        _main_kernel,
        out_shape=jax.ShapeDtypeStruct((_B, _C, _V), jnp.float32),
        grid_spec=pltpu.PrefetchScalarGridSpec(
            num_scalar_prefetch=1,
            grid=(_B, _NB),
            in_specs=[pl.BlockSpec((1, _ROWS, _NPAD),
                                   lambda b, j, s: (b, 0, 0))],
            out_specs=pl.BlockSpec((1, _C, _VB), lambda b, j, s: (b, 0, j)),
            scratch_shapes=[pltpu.VMEM((_ROWS, _VB), jnp.float32)],
        ),
        compiler_params=pltpu.CompilerParams(
            dimension_semantics=("parallel", "arbitrary"),
            vmem_limit_bytes=56 * 1024 * 1024),
    )(starts, packed)

    return out.reshape(_B, _C, _D, _H, _W)
